# baseline (speedup 1.0000x reference)
import functools

import jax
import jax.numpy as jnp
from jax import lax
from jax.experimental import pallas as pl
from jax.experimental.pallas import tpu as pltpu
from jax.experimental.pallas import tpu_sc as plsc

BATCH = 4096
SEQ = 200
EMBED = 64
HALF = SEQ // 2

NUM_CORES = 2
NUM_SUBCORES = 16
NUM_WORKERS = NUM_CORES * NUM_SUBCORES
BATCH_PER_WORKER = BATCH // NUM_WORKERS


@functools.partial(
    pl.kernel,
    out_type=jax.ShapeDtypeStruct((BATCH, 2, HALF, EMBED), jnp.float32),
    mesh=plsc.VectorSubcoreMesh(core_axis_name="c", subcore_axis_name="s"),
    compiler_params=pltpu.CompilerParams(use_tc_tiling_on_sc=False),
    scratch_types=[
        pltpu.VMEM((2, HALF), jnp.int32),
        pltpu.VMEM((2, HALF, EMBED), jnp.float32),
        pltpu.VMEM((2, HALF, EMBED), jnp.float32),
        pltpu.SemaphoreType.DMA,
    ],
)
def _sc_embed(x_hbm, tok_hbm, pos_hbm, out_hbm, idx_v, rows_v, pos_v, sem):
    wid = lax.axis_index("s") * NUM_CORES + lax.axis_index("c")
    base = wid * BATCH_PER_WORKER

    pltpu.sync_copy(pos_hbm, pos_v)

    def batch_body(i, carry):
        b = base + i
        pltpu.sync_copy(x_hbm.at[b], idx_v)
        cp0 = pltpu.async_copy(tok_hbm.at[idx_v.at[0]], rows_v.at[0], sem)
        cp1 = pltpu.async_copy(tok_hbm.at[idx_v.at[1]], rows_v.at[1], sem)
        cp0.wait()
        cp1.wait()

        def row_body(r, rcarry):
            for h in range(2):
                for c in range(EMBED // 16):
                    sl = pl.ds(c * 16, 16)
                    rows_v[h, r, sl] = rows_v[h, r, sl] + pos_v[h, r, sl]
            return rcarry

        lax.fori_loop(0, HALF, row_body, 0)
        pltpu.sync_copy(rows_v, out_hbm.at[b])
        return carry

    lax.fori_loop(0, BATCH_PER_WORKER, batch_body, 0)


def kernel(x, token_table, position_table):
    x2 = x.astype(jnp.int32).reshape(BATCH, 2, HALF)
    pos2 = position_table.reshape(2, HALF, EMBED)
    out = _sc_embed(x2, token_table, pos2)
    return out.reshape(BATCH, SEQ, EMBED)
